# baseline (speedup 1.0000x reference)
import functools

import numpy as np
import jax
import jax.numpy as jnp
from jax import lax
from jax.experimental import pallas as pl
from jax.experimental.pallas import tpu as pltpu
from jax.experimental.pallas import tpu_sc as plsc

_N = 128
_K = 512
_BS = 8192
_H = np.float32(1.0 / 127.0)
_C1 = np.float32(127.0 / 6.0)
_C2 = np.float32(1.0 / 762.0)

_NTILES = 32
_DCH = _K // _NTILES
_CHUNK = 1024
_GROUPS = _CHUNK // 16

_BS_SC = 5120
_BS_TC = _BS - _BS_SC
_BLK = 512


def _build_P() -> np.ndarray:
    n = _N
    h = 1.0 / (n - 1)
    idx = np.arange(n - 2)
    S = np.zeros((n - 2, n))
    S[idx, idx] = 6.0 / h
    S[idx, idx + 1] = -12.0 / h
    S[idx, idx + 2] = 6.0 / h
    A = (np.diag(4.0 * h * np.ones(n - 2))
         + np.diag(h * np.ones(n - 3), 1)
         + np.diag(h * np.ones(n - 3), -1))
    P = np.zeros((n, n))
    P[1:-1] = np.linalg.solve(A, S)
    return P.astype(np.float32)


_P_CONST = _build_P()


def _coeffs(t):
    tf = t * np.float32(127.0)
    ivec = jnp.clip(tf.astype(jnp.int32), 0, 126)
    fidx = ivec.astype(jnp.float32)
    x0 = fidx * _H
    x1 = (fidx + np.float32(1.0)) * _H
    dx0 = t - x0
    dx1 = x1 - t
    b = dx0 * np.float32(127.0)
    c = dx1 * (_C1 * dx1 * dx1 - _C2)
    d = dx0 * (_C1 * dx0 * dx0 - _C2)
    return ivec, b, c, d


def _mtable_body(p_ref, x_ref, t8_ref):
    x = x_ref[:]
    m = jnp.dot(p_ref[:], x, preferred_element_type=jnp.float32,
                precision=lax.Precision.HIGHEST)
    pad = lambda z: jnp.concatenate([z[1:], z[-1:]], axis=0)
    x1 = pad(x)
    m1 = pad(m)
    y, s = x[:, :_K], x[:, _K:]
    dy, ds = x1[:, :_K] - y, x1[:, _K:] - s
    my, ms = m[:, :_K], m[:, _K:]
    my1, ms1 = m1[:, :_K], m1[:, _K:]
    for k, part in enumerate([y, dy, my, my1, s, ds, ms, ms1]):
        t8_ref[k, :, :] = part


def _tc_eval_body(t_ref, yg_ref, sg_ref, mmu_ref, msg_ref, mu_ref, sig_ref):
    t = t_ref[:]
    i, b, c, d = _coeffs(t)
    a = np.float32(1.0) - b

    iota = lax.broadcasted_iota(jnp.int32, (_BLK, _N), 1)
    e0 = iota == i[:, None]
    e1 = iota == (i + 1)[:, None]
    zero = jnp.zeros((), jnp.float32)
    hmat = jnp.where(e0, a[:, None], zero) + jnp.where(e1, b[:, None], zero)
    gmat = jnp.where(e0, c[:, None], zero) + jnp.where(e1, d[:, None], zero)

    dot = lambda x, y: jnp.dot(x, y, preferred_element_type=jnp.float32,
                               precision=lax.Precision.DEFAULT)
    mu_ref[:] = dot(hmat, yg_ref[:]) + dot(gmat, mmu_ref[:])
    sig_ref[:] = jnp.exp(dot(hmat, sg_ref[:]) + dot(gmat, msg_ref[:]))


def _sc_eval_body(t_hbm, tab_hbm, mu_hbm, sg_hbm,
                  t_v, tab_v2d, mu_b0, mu_b1, sg_b0, sg_b1, sem0, sem1):
    wid = lax.axis_index("s") * 2 + lax.axis_index("c")
    pltpu.sync_copy(t_hbm, t_v)
    pltpu.sync_copy(tab_hbm.at[wid], tab_v2d)
    subtabs = [tab_v2d.at[k] for k in range(8)]
    lane = lax.iota(jnp.int32, 16)
    col0 = wid * _DCH

    mu_bufs = (mu_b0, mu_b1)
    sg_bufs = (sg_b0, sg_b1)
    sems = (sem0, sem1)
    pending = [None, None]

    for cidx in range(_BS_SC // _CHUNK):
        par = cidx % 2
        if pending[par] is not None:
            pending[par][0].wait()
            pending[par][1].wait()
        mu_buf, sg_buf, sem = mu_bufs[par], sg_bufs[par], sems[par]

        @plsc.parallel_loop(0, _GROUPS, step=1, unroll=1)
        def group_body(g, mu_buf=mu_buf, sg_buf=sg_buf, cbase=cidx * _CHUNK):
            t16 = t_v[pl.ds(cbase + g * 16, 16)]
            ivec, b16, c16, d16 = _coeffs(t16)
            base = ivec * _DCH
            for p in range(16):
                ix = jnp.broadcast_to(base[p], (16,)) + lane
                y0 = plsc.load_gather(subtabs[0], [ix])
                dy = plsc.load_gather(subtabs[1], [ix])
                m0 = plsc.load_gather(subtabs[2], [ix])
                m1 = plsc.load_gather(subtabs[3], [ix])
                s0 = plsc.load_gather(subtabs[4], [ix])
                ds = plsc.load_gather(subtabs[5], [ix])
                n0 = plsc.load_gather(subtabs[6], [ix])
                n1 = plsc.load_gather(subtabs[7], [ix])
                bv = jnp.broadcast_to(b16[p], (16,))
                cv = jnp.broadcast_to(c16[p], (16,))
                dv = jnp.broadcast_to(d16[p], (16,))
                mu = y0 + bv * dy + cv * m0 + dv * m1
                sg = jnp.exp(s0 + bv * ds + cv * n0 + dv * n1)
                row = g * 16 + p
                mu_buf[row, :] = mu
                sg_buf[row, :] = sg

        rows = pl.ds(cidx * _CHUNK, _CHUNK)
        cols = pl.ds(col0, _DCH)
        cp_mu = pltpu.async_copy(mu_buf, mu_hbm.at[rows, cols], sem)
        cp_sg = pltpu.async_copy(sg_buf, sg_hbm.at[rows, cols], sem)
        pending[par] = (cp_mu, cp_sg)

    for par in range(2):
        if pending[par] is not None:
            pending[par][0].wait()
            pending[par][1].wait()


_sc_eval = functools.partial(
    pl.kernel,
    out_type=[jax.ShapeDtypeStruct((_BS_SC, _K), jnp.float32),
              jax.ShapeDtypeStruct((_BS_SC, _K), jnp.float32)],
    mesh=plsc.VectorSubcoreMesh(core_axis_name="c", subcore_axis_name="s"),
    scratch_types=[
        pltpu.VMEM((_BS_SC,), jnp.float32),
        pltpu.VMEM((8, _N * _DCH), jnp.float32),
        pltpu.VMEM((_CHUNK, _DCH), jnp.float32),
        pltpu.VMEM((_CHUNK, _DCH), jnp.float32),
        pltpu.VMEM((_CHUNK, _DCH), jnp.float32),
        pltpu.VMEM((_CHUNK, _DCH), jnp.float32),
        pltpu.SemaphoreType.DMA,
        pltpu.SemaphoreType.DMA,
    ],
    compiler_params=pltpu.CompilerParams(use_tc_tiling_on_sc=False,
                                         needs_layout_passes=False),
)(_sc_eval_body)


def kernel(t, mu_params, sigma_params, w_logits):
    ones_row = jnp.ones((1, _K), jnp.float32)
    y_grid = jnp.concatenate([-ones_row, mu_params, ones_row], axis=0)
    s_grid = jnp.concatenate([0.0 * ones_row, sigma_params, 0.0 * ones_row], axis=0)

    p_const = jnp.asarray(_P_CONST)
    x_both = jnp.concatenate([y_grid, s_grid], axis=1)
    t8 = pl.pallas_call(
        _mtable_body,
        out_shape=jax.ShapeDtypeStruct((8, _N, _K), jnp.float32),
    )(p_const, x_both)

    tab = t8.reshape(8, _N, _NTILES, _DCH).transpose(2, 0, 1, 3)
    tab = tab.reshape(_NTILES, 8, _N * _DCH)

    mu_sc, sig_sc = _sc_eval(t[_BS_TC:], tab)

    full = pl.BlockSpec((_N, _K), lambda bidx: (0, 0))
    mu_tc, sig_tc = pl.pallas_call(
        _tc_eval_body,
        grid=(_BS_TC // _BLK,),
        in_specs=[
            pl.BlockSpec((_BLK,), lambda bidx: (bidx,)),
            full, full, full, full,
        ],
        out_specs=[
            pl.BlockSpec((_BLK, _K), lambda bidx: (bidx, 0)),
            pl.BlockSpec((_BLK, _K), lambda bidx: (bidx, 0)),
        ],
        out_shape=[
            jax.ShapeDtypeStruct((_BS_TC, _K), jnp.float32),
            jax.ShapeDtypeStruct((_BS_TC, _K), jnp.float32),
        ],
    )(t[:_BS_TC], t8[0], t8[4], t8[2], t8[6])

    mu = jnp.concatenate([mu_tc, mu_sc], axis=0)
    sig = jnp.concatenate([sig_tc, sig_sc], axis=0)
    return (mu.reshape(_BS, 8, 64), sig.reshape(_BS, 8, 64), w_logits)


# speedup vs baseline: 1.0572x; 1.0572x over previous
import functools

import numpy as np
import jax
import jax.numpy as jnp
from jax import lax
from jax.experimental import pallas as pl
from jax.experimental.pallas import tpu as pltpu
from jax.experimental.pallas import tpu_sc as plsc

_N = 128
_K = 512
_BS = 8192
_H = np.float32(1.0 / 127.0)
_C1 = np.float32(127.0 / 6.0)
_C2 = np.float32(1.0 / 762.0)

_NTILES = 32
_DCH = _K // _NTILES
_CHUNK = 1024
_GROUPS = _CHUNK // 16

_BS_SC = 4096
_BS_TC = _BS - _BS_SC
_BLK = 512


def _build_P() -> np.ndarray:
    n = _N
    h = 1.0 / (n - 1)
    idx = np.arange(n - 2)
    S = np.zeros((n - 2, n))
    S[idx, idx] = 6.0 / h
    S[idx, idx + 1] = -12.0 / h
    S[idx, idx + 2] = 6.0 / h
    A = (np.diag(4.0 * h * np.ones(n - 2))
         + np.diag(h * np.ones(n - 3), 1)
         + np.diag(h * np.ones(n - 3), -1))
    P = np.zeros((n, n))
    P[1:-1] = np.linalg.solve(A, S)
    return P.astype(np.float32)


_P_CONST = _build_P()


def _coeffs(t):
    tf = t * np.float32(127.0)
    ivec = jnp.clip(tf.astype(jnp.int32), 0, 126)
    fidx = ivec.astype(jnp.float32)
    x0 = fidx * _H
    x1 = (fidx + np.float32(1.0)) * _H
    dx0 = t - x0
    dx1 = x1 - t
    b = dx0 * np.float32(127.0)
    c = dx1 * (_C1 * dx1 * dx1 - _C2)
    d = dx0 * (_C1 * dx0 * dx0 - _C2)
    return ivec, b, c, d


def _mtable_body(p_ref, x_ref, t8_ref):
    x = x_ref[:]
    m = jnp.dot(p_ref[:], x, preferred_element_type=jnp.float32,
                precision=lax.Precision.HIGHEST)
    pad = lambda z: jnp.concatenate([z[1:], z[-1:]], axis=0)
    x1 = pad(x)
    m1 = pad(m)
    y, s = x[:, :_K], x[:, _K:]
    dy, ds = x1[:, :_K] - y, x1[:, _K:] - s
    my, ms = m[:, :_K], m[:, _K:]
    my1, ms1 = m1[:, :_K], m1[:, _K:]
    for k, part in enumerate([y, dy, my, my1, s, ds, ms, ms1]):
        t8_ref[k, :, :] = part


def _tc_eval_body(t_ref, yg_ref, sg_ref, mmu_ref, msg_ref, mu_ref, sig_ref):
    t = t_ref[:]
    i, b, c, d = _coeffs(t)
    a = np.float32(1.0) - b

    iota = lax.broadcasted_iota(jnp.int32, (_BLK, _N), 1)
    e0 = iota == i[:, None]
    e1 = iota == (i + 1)[:, None]
    zero = jnp.zeros((), jnp.float32)
    hmat = jnp.where(e0, a[:, None], zero) + jnp.where(e1, b[:, None], zero)
    gmat = jnp.where(e0, c[:, None], zero) + jnp.where(e1, d[:, None], zero)

    dot = lambda x, y: jnp.dot(x, y, preferred_element_type=jnp.float32,
                               precision=lax.Precision.DEFAULT)
    mu_ref[:] = dot(hmat, yg_ref[:]) + dot(gmat, mmu_ref[:])
    sig_ref[:] = jnp.exp(dot(hmat, sg_ref[:]) + dot(gmat, msg_ref[:]))


def _sc_eval_body(t_hbm, tab_hbm, mu_hbm, sg_hbm,
                  t_v, tab_v2d, mu_b0, mu_b1, sg_b0, sg_b1, sem0, sem1):
    wid = lax.axis_index("s") * 2 + lax.axis_index("c")
    pltpu.sync_copy(t_hbm, t_v)
    pltpu.sync_copy(tab_hbm.at[wid], tab_v2d)
    subtabs = [tab_v2d.at[k] for k in range(8)]
    lane = lax.iota(jnp.int32, 16)
    col0 = wid * _DCH

    mu_bufs = (mu_b0, mu_b1)
    sg_bufs = (sg_b0, sg_b1)
    sems = (sem0, sem1)
    pending = [None, None]

    for cidx in range(_BS_SC // _CHUNK):
        par = cidx % 2
        if pending[par] is not None:
            pending[par][0].wait()
            pending[par][1].wait()
        mu_buf, sg_buf, sem = mu_bufs[par], sg_bufs[par], sems[par]

        @plsc.parallel_loop(0, _GROUPS, step=1, unroll=1)
        def group_body(g, mu_buf=mu_buf, sg_buf=sg_buf, cbase=cidx * _CHUNK):
            t16 = t_v[pl.ds(cbase + g * 16, 16)]
            ivec, b16, c16, d16 = _coeffs(t16)
            base = ivec * _DCH
            for p in range(16):
                ix = jnp.broadcast_to(base[p], (16,)) + lane
                y0 = plsc.load_gather(subtabs[0], [ix])
                dy = plsc.load_gather(subtabs[1], [ix])
                m0 = plsc.load_gather(subtabs[2], [ix])
                m1 = plsc.load_gather(subtabs[3], [ix])
                s0 = plsc.load_gather(subtabs[4], [ix])
                ds = plsc.load_gather(subtabs[5], [ix])
                n0 = plsc.load_gather(subtabs[6], [ix])
                n1 = plsc.load_gather(subtabs[7], [ix])
                bv = jnp.broadcast_to(b16[p], (16,))
                cv = jnp.broadcast_to(c16[p], (16,))
                dv = jnp.broadcast_to(d16[p], (16,))
                mu = y0 + bv * dy + cv * m0 + dv * m1
                sg = jnp.exp(s0 + bv * ds + cv * n0 + dv * n1)
                row = g * 16 + p
                mu_buf[row, :] = mu
                sg_buf[row, :] = sg

        rows = pl.ds(cidx * _CHUNK, _CHUNK)
        cols = pl.ds(col0, _DCH)
        cp_mu = pltpu.async_copy(mu_buf, mu_hbm.at[rows, cols], sem)
        cp_sg = pltpu.async_copy(sg_buf, sg_hbm.at[rows, cols], sem)
        pending[par] = (cp_mu, cp_sg)

    for par in range(2):
        if pending[par] is not None:
            pending[par][0].wait()
            pending[par][1].wait()


_sc_eval = functools.partial(
    pl.kernel,
    out_type=[jax.ShapeDtypeStruct((_BS_SC, _K), jnp.float32),
              jax.ShapeDtypeStruct((_BS_SC, _K), jnp.float32)],
    mesh=plsc.VectorSubcoreMesh(core_axis_name="c", subcore_axis_name="s"),
    scratch_types=[
        pltpu.VMEM((_BS_SC,), jnp.float32),
        pltpu.VMEM((8, _N * _DCH), jnp.float32),
        pltpu.VMEM((_CHUNK, _DCH), jnp.float32),
        pltpu.VMEM((_CHUNK, _DCH), jnp.float32),
        pltpu.VMEM((_CHUNK, _DCH), jnp.float32),
        pltpu.VMEM((_CHUNK, _DCH), jnp.float32),
        pltpu.SemaphoreType.DMA,
        pltpu.SemaphoreType.DMA,
    ],
    compiler_params=pltpu.CompilerParams(use_tc_tiling_on_sc=False,
                                         needs_layout_passes=False),
)(_sc_eval_body)


def kernel(t, mu_params, sigma_params, w_logits):
    ones_row = jnp.ones((1, _K), jnp.float32)
    y_grid = jnp.concatenate([-ones_row, mu_params, ones_row], axis=0)
    s_grid = jnp.concatenate([0.0 * ones_row, sigma_params, 0.0 * ones_row], axis=0)

    p_const = jnp.asarray(_P_CONST)
    x_both = jnp.concatenate([y_grid, s_grid], axis=1)
    t8 = pl.pallas_call(
        _mtable_body,
        out_shape=jax.ShapeDtypeStruct((8, _N, _K), jnp.float32),
    )(p_const, x_both)

    tab = t8.reshape(8, _N, _NTILES, _DCH).transpose(2, 0, 1, 3)
    tab = tab.reshape(_NTILES, 8, _N * _DCH)

    mu_sc, sig_sc = _sc_eval(t[_BS_TC:], tab)

    full = pl.BlockSpec((_N, _K), lambda bidx: (0, 0))
    mu_tc, sig_tc = pl.pallas_call(
        _tc_eval_body,
        grid=(_BS_TC // _BLK,),
        in_specs=[
            pl.BlockSpec((_BLK,), lambda bidx: (bidx,)),
            full, full, full, full,
        ],
        out_specs=[
            pl.BlockSpec((_BLK, _K), lambda bidx: (bidx, 0)),
            pl.BlockSpec((_BLK, _K), lambda bidx: (bidx, 0)),
        ],
        out_shape=[
            jax.ShapeDtypeStruct((_BS_TC, _K), jnp.float32),
            jax.ShapeDtypeStruct((_BS_TC, _K), jnp.float32),
        ],
    )(t[:_BS_TC], t8[0], t8[4], t8[2], t8[6])

    mu = jnp.concatenate([mu_tc, mu_sc], axis=0)
    sig = jnp.concatenate([sig_tc, sig_sc], axis=0)
    return (mu.reshape(_BS, 8, 64), sig.reshape(_BS, 8, 64), w_logits)


# speedup vs baseline: 1.0818x; 1.0232x over previous
import functools

import numpy as np
import jax
import jax.numpy as jnp
from jax import lax
from jax.experimental import pallas as pl
from jax.experimental.pallas import tpu as pltpu
from jax.experimental.pallas import tpu_sc as plsc

_N = 128
_K = 512
_BS = 8192
_H = np.float32(1.0 / 127.0)
_C1 = np.float32(127.0 / 6.0)
_C2 = np.float32(1.0 / 762.0)

_NTILES = 32
_DCH = _K // _NTILES
_CHUNK = 1024
_GROUPS = _CHUNK // 16

_BS_SC = 4096
_BS_TC = _BS - _BS_SC
_BLK = 512


def _build_P() -> np.ndarray:
    n = _N
    h = 1.0 / (n - 1)
    idx = np.arange(n - 2)
    S = np.zeros((n - 2, n))
    S[idx, idx] = 6.0 / h
    S[idx, idx + 1] = -12.0 / h
    S[idx, idx + 2] = 6.0 / h
    A = (np.diag(4.0 * h * np.ones(n - 2))
         + np.diag(h * np.ones(n - 3), 1)
         + np.diag(h * np.ones(n - 3), -1))
    P = np.zeros((n, n))
    P[1:-1] = np.linalg.solve(A, S)
    return P.astype(np.float32)


_P_CONST = _build_P()


def _coeffs(t):
    tf = t * np.float32(127.0)
    ivec = jnp.clip(tf.astype(jnp.int32), 0, 126)
    fidx = ivec.astype(jnp.float32)
    x0 = fidx * _H
    x1 = (fidx + np.float32(1.0)) * _H
    dx0 = t - x0
    dx1 = x1 - t
    b = dx0 * np.float32(127.0)
    c = dx1 * (_C1 * dx1 * dx1 - _C2)
    d = dx0 * (_C1 * dx0 * dx0 - _C2)
    return ivec, b, c, d


def _mtable_body(p_ref, x_ref, t8_ref):
    x = x_ref[:]
    m = jnp.dot(p_ref[:], x, preferred_element_type=jnp.float32,
                precision=lax.Precision.HIGHEST)
    pad = lambda z: jnp.concatenate([z[1:], z[-1:]], axis=0)
    x1 = pad(x)
    m1 = pad(m)
    y, s = x[:, :_K], x[:, _K:]
    dy, ds = x1[:, :_K] - y, x1[:, _K:] - s
    my, ms = m[:, :_K], m[:, _K:]
    my1, ms1 = m1[:, :_K], m1[:, _K:]
    for k, part in enumerate([y, dy, my, my1, s, ds, ms, ms1]):
        t8_ref[k, :, :] = part


def _tc_eval_body(t_ref, yg_ref, sg_ref, mmu_ref, msg_ref, mu_ref, sig_ref):
    t = t_ref[:]
    i, b, c, d = _coeffs(t)
    a = np.float32(1.0) - b

    iota = lax.broadcasted_iota(jnp.int32, (_BLK, _N), 1)
    e0 = iota == i[:, None]
    e1 = iota == (i + 1)[:, None]
    zero = jnp.zeros((), jnp.float32)
    hmat = jnp.where(e0, a[:, None], zero) + jnp.where(e1, b[:, None], zero)
    gmat = jnp.where(e0, c[:, None], zero) + jnp.where(e1, d[:, None], zero)

    dot = lambda x, y: jnp.dot(x, y, preferred_element_type=jnp.float32,
                               precision=lax.Precision.HIGHEST)
    mu_ref[:] = dot(hmat, yg_ref[:]) + dot(gmat, mmu_ref[:])
    sig_ref[:] = jnp.exp(dot(hmat, sg_ref[:]) + dot(gmat, msg_ref[:]))


def _sc_eval_body(t_hbm, tab_hbm, mu_hbm, sg_hbm,
                  t_v, tab_v2d, mu_b0, mu_b1, sg_b0, sg_b1, sem0, sem1):
    wid = lax.axis_index("s") * 2 + lax.axis_index("c")
    pltpu.sync_copy(t_hbm, t_v)
    pltpu.sync_copy(tab_hbm.at[wid], tab_v2d)
    subtabs = [tab_v2d.at[k] for k in range(8)]
    lane = lax.iota(jnp.int32, 16)
    col0 = wid * _DCH

    mu_bufs = (mu_b0, mu_b1)
    sg_bufs = (sg_b0, sg_b1)
    sems = (sem0, sem1)
    pending = [None, None]

    for cidx in range(_BS_SC // _CHUNK):
        par = cidx % 2
        if pending[par] is not None:
            pending[par][0].wait()
            pending[par][1].wait()
        mu_buf, sg_buf, sem = mu_bufs[par], sg_bufs[par], sems[par]

        @plsc.parallel_loop(0, _GROUPS, step=1, unroll=1)
        def group_body(g, mu_buf=mu_buf, sg_buf=sg_buf, cbase=cidx * _CHUNK):
            t16 = t_v[pl.ds(cbase + g * 16, 16)]
            ivec, b16, c16, d16 = _coeffs(t16)
            base = ivec * _DCH
            for p in range(16):
                ix = jnp.broadcast_to(base[p], (16,)) + lane
                y0 = plsc.load_gather(subtabs[0], [ix])
                dy = plsc.load_gather(subtabs[1], [ix])
                m0 = plsc.load_gather(subtabs[2], [ix])
                m1 = plsc.load_gather(subtabs[3], [ix])
                s0 = plsc.load_gather(subtabs[4], [ix])
                ds = plsc.load_gather(subtabs[5], [ix])
                n0 = plsc.load_gather(subtabs[6], [ix])
                n1 = plsc.load_gather(subtabs[7], [ix])
                bv = jnp.broadcast_to(b16[p], (16,))
                cv = jnp.broadcast_to(c16[p], (16,))
                dv = jnp.broadcast_to(d16[p], (16,))
                mu = y0 + bv * dy + cv * m0 + dv * m1
                sg = jnp.exp(s0 + bv * ds + cv * n0 + dv * n1)
                row = g * 16 + p
                mu_buf[row, :] = mu
                sg_buf[row, :] = sg

        rows = pl.ds(cidx * _CHUNK, _CHUNK)
        cols = pl.ds(col0, _DCH)
        cp_mu = pltpu.async_copy(mu_buf, mu_hbm.at[rows, cols], sem)
        cp_sg = pltpu.async_copy(sg_buf, sg_hbm.at[rows, cols], sem)
        pending[par] = (cp_mu, cp_sg)

    for par in range(2):
        if pending[par] is not None:
            pending[par][0].wait()
            pending[par][1].wait()


_sc_eval = functools.partial(
    pl.kernel,
    out_type=[jax.ShapeDtypeStruct((_BS_SC, _K), jnp.float32),
              jax.ShapeDtypeStruct((_BS_SC, _K), jnp.float32)],
    mesh=plsc.VectorSubcoreMesh(core_axis_name="c", subcore_axis_name="s"),
    scratch_types=[
        pltpu.VMEM((_BS_SC,), jnp.float32),
        pltpu.VMEM((8, _N * _DCH), jnp.float32),
        pltpu.VMEM((_CHUNK, _DCH), jnp.float32),
        pltpu.VMEM((_CHUNK, _DCH), jnp.float32),
        pltpu.VMEM((_CHUNK, _DCH), jnp.float32),
        pltpu.VMEM((_CHUNK, _DCH), jnp.float32),
        pltpu.SemaphoreType.DMA,
        pltpu.SemaphoreType.DMA,
    ],
    compiler_params=pltpu.CompilerParams(use_tc_tiling_on_sc=False,
                                         needs_layout_passes=False),
)(_sc_eval_body)


def kernel(t, mu_params, sigma_params, w_logits):
    ones_row = jnp.ones((1, _K), jnp.float32)
    y_grid = jnp.concatenate([-ones_row, mu_params, ones_row], axis=0)
    s_grid = jnp.concatenate([0.0 * ones_row, sigma_params, 0.0 * ones_row], axis=0)

    p_const = jnp.asarray(_P_CONST)
    x_both = jnp.concatenate([y_grid, s_grid], axis=1)
    t8 = pl.pallas_call(
        _mtable_body,
        out_shape=jax.ShapeDtypeStruct((8, _N, _K), jnp.float32),
    )(p_const, x_both)

    tab = t8.reshape(8, _N, _NTILES, _DCH).transpose(2, 0, 1, 3)
    tab = tab.reshape(_NTILES, 8, _N * _DCH)

    mu_sc, sig_sc = _sc_eval(t[_BS_TC:], tab)

    full = pl.BlockSpec((_N, _K), lambda bidx: (0, 0))
    mu_tc, sig_tc = pl.pallas_call(
        _tc_eval_body,
        grid=(_BS_TC // _BLK,),
        in_specs=[
            pl.BlockSpec((_BLK,), lambda bidx: (bidx,)),
            full, full, full, full,
        ],
        out_specs=[
            pl.BlockSpec((_BLK, _K), lambda bidx: (bidx, 0)),
            pl.BlockSpec((_BLK, _K), lambda bidx: (bidx, 0)),
        ],
        out_shape=[
            jax.ShapeDtypeStruct((_BS_TC, _K), jnp.float32),
            jax.ShapeDtypeStruct((_BS_TC, _K), jnp.float32),
        ],
    )(t[:_BS_TC], t8[0], t8[4], t8[2], t8[6])

    mu = jnp.concatenate([mu_tc, mu_sc], axis=0)
    sig = jnp.concatenate([sig_tc, sig_sc], axis=0)
    return (mu.reshape(_BS, 8, 64), sig.reshape(_BS, 8, 64), w_logits)
